# baseline (speedup 1.0000x reference)
import functools

import jax
import jax.numpy as jnp
from jax import lax
from jax.experimental import pallas as pl
from jax.experimental.pallas import tpu as pltpu
from jax.experimental.pallas import tpu_sc as plsc

B = 4096
L = 200
D = 64
EPS = 1e-5
POOL = 203

NC = 2
NS = 16
NW = NC * NS
BPW = B // NW

LHALF = L // 2


def _sc_pool_kernel(plates_hbm, adv_hbm, sig_hbm, yr_hbm,
                    ptab_hbm, atab_hbm, stab_hbm, ytab_hbm,
                    out_hbm,
                    pidx_v, rows_v, sidx_v, small_v, out_v, sem):
    wid = lax.axis_index("s") * NC + lax.axis_index("c")
    base = wid * BPW

    pltpu.sync_copy(adv_hbm.at[pl.ds(base, BPW)], sidx_v)
    pltpu.async_copy(atab_hbm.at[sidx_v], small_v, sem).wait()

    @pl.loop(0, BPW)
    def _(r):
        for c in range(D // 16):
            out_v[r, pl.ds(16 * c, 16)] = small_v[r, pl.ds(16 * c, 16)]

    pltpu.sync_copy(sig_hbm.at[pl.ds(base, BPW)], sidx_v)
    pltpu.async_copy(stab_hbm.at[sidx_v], small_v, sem).wait()

    @pl.loop(0, BPW)
    def _(r):
        for c in range(D // 16):
            out_v[r, pl.ds(16 * c, 16)] += small_v[r, pl.ds(16 * c, 16)]

    pltpu.sync_copy(yr_hbm.at[pl.ds(base, BPW)], sidx_v)
    pltpu.async_copy(ytab_hbm.at[sidx_v], small_v, sem).wait()

    @pl.loop(0, BPW)
    def _(r):
        for c in range(D // 16):
            out_v[r, pl.ds(16 * c, 16)] += small_v[r, pl.ds(16 * c, 16)]

    @pl.loop(0, BPW)
    def _(i):
        pltpu.sync_copy(plates_hbm.at[base + i], pidx_v)
        cp0 = pltpu.make_async_copy(
            ptab_hbm.at[pidx_v.at[0]], rows_v.at[pl.ds(0, LHALF)], sem)
        cp1 = pltpu.make_async_copy(
            ptab_hbm.at[pidx_v.at[1]], rows_v.at[pl.ds(LHALF, LHALF)], sem)
        cp0.start()
        cp1.start()
        cp0.wait()
        cp1.wait()

        def body(j, acc):
            a0, a1, a2, a3 = acc
            a0 = a0 + rows_v[j, pl.ds(0, 16)]
            a1 = a1 + rows_v[j, pl.ds(16, 16)]
            a2 = a2 + rows_v[j, pl.ds(32, 16)]
            a3 = a3 + rows_v[j, pl.ds(48, 16)]
            return (a0, a1, a2, a3)

        z = jnp.zeros((16,), jnp.float32)
        a0, a1, a2, a3 = lax.fori_loop(0, L, body, (z, z, z, z))
        out_v[i, pl.ds(0, 16)] += a0
        out_v[i, pl.ds(16, 16)] += a1
        out_v[i, pl.ds(32, 16)] += a2
        out_v[i, pl.ds(48, 16)] += a3

    pltpu.sync_copy(out_v, out_hbm.at[pl.ds(base, BPW)])


def _sc_pool(plates, adv, sig, yr, ptab, atab, stab, ytab):
    mesh = plsc.VectorSubcoreMesh(core_axis_name="c", subcore_axis_name="s")
    kern = pl.kernel(
        _sc_pool_kernel,
        out_type=jax.ShapeDtypeStruct((B, D), jnp.float32),
        mesh=mesh,
        compiler_params=pltpu.CompilerParams(use_tc_tiling_on_sc=False),
        scratch_types=[
            pltpu.VMEM((2, LHALF), jnp.int32),
            pltpu.VMEM((L, D), jnp.float32),
            pltpu.VMEM((BPW,), jnp.int32),
            pltpu.VMEM((BPW, D), jnp.float32),
            pltpu.VMEM((BPW, D), jnp.float32),
            pltpu.SemaphoreType.DMA,
        ],
    )
    return kern(plates.reshape(B, 2, LHALF), adv, sig, yr,
                ptab, atab, stab, ytab)


def _tc_mlp_kernel(x_ref, lng_ref, lnb_ref, w1_ref, b1_ref, w2_ref, b2_ref,
                   w3_ref, b3_ref, out_ref):
    x = x_ref[...] * (1.0 / POOL)
    mu = jnp.mean(x, axis=1, keepdims=True)
    xc = x - mu
    var = jnp.mean(xc * xc, axis=1, keepdims=True)
    x = xc * lax.rsqrt(var + EPS) * lng_ref[...] + lnb_ref[...]
    h = jnp.dot(x, w1_ref[...], preferred_element_type=jnp.float32)
    h = jnp.maximum(h + b1_ref[...], 0.0)
    h = jnp.dot(h, w2_ref[...], preferred_element_type=jnp.float32)
    h = jnp.maximum(h + b2_ref[...], 0.0)
    out_ref[...] = (
        jnp.dot(h, w3_ref[...], preferred_element_type=jnp.float32)
        + b3_ref[...])


def _tc_mlp(pooled, ln_g, ln_b, W1, b1, W2, b2, W3, b3):
    return pl.pallas_call(
        _tc_mlp_kernel,
        out_shape=jax.ShapeDtypeStruct((B, 1), jnp.float32),
    )(pooled, ln_g.reshape(1, D), ln_b.reshape(1, D),
      W1, b1.reshape(1, 128), W2, b2.reshape(1, 64), W3, b3.reshape(1, 1))


@jax.jit
def kernel(plates, advantages_on_road, significances, years, plate_table,
           adv_table, sig_table, year_table, ln_g, ln_b, W1, b1, W2, b2,
           W3, b3):
    pooled = _sc_pool(plates, advantages_on_road, significances, years,
                      plate_table, adv_table, sig_table, year_table)
    return _tc_mlp(pooled, ln_g, ln_b, W1, b1, W2, b2, W3, b3)


# speedup vs baseline: 1.2742x; 1.2742x over previous
import jax
import jax.numpy as jnp
from jax import lax
from jax.experimental import pallas as pl
from jax.experimental.pallas import tpu as pltpu
from jax.experimental.pallas import tpu_sc as plsc

B = 4096
L = 200
D = 64
EPS = 1e-5
POOL = 203

NC = 2
NS = 16
NW = NC * NS
BPW = B // NW

LHALF = L // 2
NBUF = 4


def _sc_pool_kernel(plates_hbm, adv_hbm, sig_hbm, yr_hbm,
                    ptab_hbm, atab_hbm, stab_hbm, ytab_hbm,
                    out_hbm,
                    pidx_all, sidx_v, small_v, out_v, sem0, *ring):
    bufs = ring[:NBUF]
    sems = ring[NBUF:]
    wid = lax.axis_index("s") * NC + lax.axis_index("c")
    base = wid * BPW

    def add_small(idx_hbm, tab_hbm, first):
        pltpu.sync_copy(idx_hbm.at[pl.ds(base, BPW)], sidx_v)
        pltpu.async_copy(tab_hbm.at[sidx_v], small_v, sem0).wait()

        @pl.loop(0, BPW)
        def _(r):
            for c in range(D // 16):
                sl = (r, pl.ds(16 * c, 16))
                if first:
                    out_v[sl] = small_v[sl]
                else:
                    out_v[sl] += small_v[sl]

    add_small(adv_hbm, atab_hbm, True)
    add_small(sig_hbm, stab_hbm, False)
    add_small(yr_hbm, ytab_hbm, False)

    pltpu.sync_copy(plates_hbm.at[pl.ds(wid * 2 * BPW, 2 * BPW)], pidx_all)

    def fire(row, b):
        pltpu.make_async_copy(
            ptab_hbm.at[pidx_all.at[2 * row]],
            bufs[b].at[pl.ds(0, LHALF)], sems[b]).start()
        pltpu.make_async_copy(
            ptab_hbm.at[pidx_all.at[2 * row + 1]],
            bufs[b].at[pl.ds(LHALF, LHALF)], sems[b]).start()

    def drain(b):
        pltpu.make_async_copy(
            ptab_hbm.at[pl.ds(0, L)], bufs[b], sems[b]).wait()

    def accum(row, b):
        buf = bufs[b]

        def body(j, acc):
            a0, a1, a2, a3 = acc
            a0 = a0 + buf[j, pl.ds(0, 16)]
            a1 = a1 + buf[j, pl.ds(16, 16)]
            a2 = a2 + buf[j, pl.ds(32, 16)]
            a3 = a3 + buf[j, pl.ds(48, 16)]
            return (a0, a1, a2, a3)

        z = jnp.zeros((16,), jnp.float32)
        a0, a1, a2, a3 = lax.fori_loop(0, L, body, (z, z, z, z), unroll=4)
        out_v[row, pl.ds(0, 16)] += a0
        out_v[row, pl.ds(16, 16)] += a1
        out_v[row, pl.ds(32, 16)] += a2
        out_v[row, pl.ds(48, 16)] += a3

    for b in range(NBUF):
        fire(b, b)

    @pl.loop(0, BPW - NBUF, step=NBUF)
    def _(i):
        for b in range(NBUF):
            drain(b)
            accum(i + b, b)
            fire(i + b + NBUF, b)

    for b in range(NBUF):
        drain(b)
        accum(BPW - NBUF + b, b)

    pltpu.sync_copy(out_v, out_hbm.at[pl.ds(base, BPW)])


def _sc_pool(plates, adv, sig, yr, ptab, atab, stab, ytab):
    mesh = plsc.VectorSubcoreMesh(core_axis_name="c", subcore_axis_name="s")
    kern = pl.kernel(
        _sc_pool_kernel,
        out_type=jax.ShapeDtypeStruct((B, D), jnp.float32),
        mesh=mesh,
        compiler_params=pltpu.CompilerParams(use_tc_tiling_on_sc=False),
        scratch_types=[
            pltpu.VMEM((2 * BPW, LHALF), jnp.int32),
            pltpu.VMEM((BPW,), jnp.int32),
            pltpu.VMEM((BPW, D), jnp.float32),
            pltpu.VMEM((BPW, D), jnp.float32),
            pltpu.SemaphoreType.DMA,
        ] + [pltpu.VMEM((L, D), jnp.float32) for _ in range(NBUF)]
          + [pltpu.SemaphoreType.DMA for _ in range(NBUF)],
    )
    return kern(plates.reshape(2 * B, LHALF), adv, sig, yr,
                ptab, atab, stab, ytab)


def _tc_mlp_kernel(x_ref, lng_ref, lnb_ref, w1_ref, b1_ref, w2_ref, b2_ref,
                   w3_ref, b3_ref, out_ref):
    x = x_ref[...] * (1.0 / POOL)
    mu = jnp.mean(x, axis=1, keepdims=True)
    xc = x - mu
    var = jnp.mean(xc * xc, axis=1, keepdims=True)
    x = xc * lax.rsqrt(var + EPS) * lng_ref[...] + lnb_ref[...]
    h = jnp.dot(x, w1_ref[...], preferred_element_type=jnp.float32)
    h = jnp.maximum(h + b1_ref[...], 0.0)
    h = jnp.dot(h, w2_ref[...], preferred_element_type=jnp.float32)
    h = jnp.maximum(h + b2_ref[...], 0.0)
    out_ref[...] = (
        jnp.dot(h, w3_ref[...], preferred_element_type=jnp.float32)
        + b3_ref[...])


def _tc_mlp(pooled, ln_g, ln_b, W1, b1, W2, b2, W3, b3):
    return pl.pallas_call(
        _tc_mlp_kernel,
        out_shape=jax.ShapeDtypeStruct((B, 1), jnp.float32),
    )(pooled, ln_g.reshape(1, D), ln_b.reshape(1, D),
      W1, b1.reshape(1, 128), W2, b2.reshape(1, 64), W3, b3.reshape(1, 1))


@jax.jit
def kernel(plates, advantages_on_road, significances, years, plate_table,
           adv_table, sig_table, year_table, ln_g, ln_b, W1, b1, W2, b2,
           W3, b3):
    pooled = _sc_pool(plates, advantages_on_road, significances, years,
                      plate_table, adv_table, sig_table, year_table)
    return _tc_mlp(pooled, ln_g, ln_b, W1, b1, W2, b2, W3, b3)
